# baseline (speedup 1.0000x reference)
import functools

import jax
import jax.numpy as jnp
from jax import lax
from jax.experimental import pallas as pl
from jax.experimental.pallas import tpu as pltpu
from jax.experimental.pallas import tpu_sc as plsc

N = 100000
E = 1600000
D_N = 27
H_IN = 8
EMB = 32
HW = 16
NP = N // 8

NTILES = 16
NCORES = 2
ROWS_PER_TILE = N // NTILES

S = 625
NSUB = 1
G = S * NSUB
NG = E // (NTILES * G)
NG_HALF = NG // 2


def _make_segsum(edge_split: bool):
    ns = NG_HALF if edge_split else NG
    goff = 1 if edge_split else 0
    mesh = plsc.VectorSubcoreMesh(core_axis_name="c", subcore_axis_name="s")

    @functools.partial(
        pl.kernel,
        out_type=jax.ShapeDtypeStruct((NCORES * N, HW), jnp.float32),
        mesh=mesh,
        compiler_params=pltpu.CompilerParams(use_tc_tiling_on_sc=False),
        scratch_types=[
            pltpu.VMEM((2, NSUB, S), jnp.int32),
            pltpu.VMEM((2, NSUB, S), jnp.int32),
            pltpu.VMEM((2, G, HW), jnp.float32),
            pltpu.MemorySpace.VMEM_SHARED((N, HW), jnp.float32),
            pltpu.SemaphoreType.DMA,
            pltpu.SemaphoreType.DMA,
        ],
    )
    def seg(t0, t1, er, out, sidx, didx, rows, acc, gsem, ssem):
        c = lax.axis_index("c")
        s = lax.axis_index("s")
        base = s * ROWS_PER_TILE

        zvec = jnp.zeros((16,), jnp.float32)

        def zero_row(i, carry):
            rows[0, i, :] = zvec
            return carry

        lax.fori_loop(0, G, zero_row, 0)
        for k in range(ROWS_PER_TILE // G):
            pltpu.sync_copy(rows.at[0], acc.at[pl.ds(base + k * G, G)])
        rem = ROWS_PER_TILE % G
        if rem:
            pltpu.sync_copy(rows.at[0, pl.ds(0, rem)],
                            acc.at[pl.ds(base + ROWS_PER_TILE - rem, rem)])
        plsc.subcore_barrier()

        def run_edges(tref):
            g0 = c * (ns * goff)

            def pair(p, carry):
                g = g0 + 2 * p
                pltpu.sync_copy(er.at[0, s, g], sidx.at[0])
                pltpu.sync_copy(er.at[1, s, g], didx.at[0])
                gh0 = [
                    pltpu.async_copy(tref.at[sidx.at[0, j]],
                                     rows.at[0, pl.ds(j * S, S)], gsem)
                    for j in range(NSUB)
                ]
                pltpu.sync_copy(er.at[0, s, g + 1], sidx.at[1])
                pltpu.sync_copy(er.at[1, s, g + 1], didx.at[1])
                gh1 = [
                    pltpu.async_copy(tref.at[sidx.at[1, j]],
                                     rows.at[1, pl.ds(j * S, S)], gsem)
                    for j in range(NSUB)
                ]
                for h in gh0:
                    h.wait()
                sh0 = [
                    pltpu.async_copy(rows.at[0, pl.ds(j * S, S)],
                                     acc.at[didx.at[0, j]], ssem, add=True)
                    for j in range(NSUB)
                ]
                for h in gh1:
                    h.wait()
                sh1 = [
                    pltpu.async_copy(rows.at[1, pl.ds(j * S, S)],
                                     acc.at[didx.at[1, j]], ssem, add=True)
                    for j in range(NSUB)
                ]
                for h in sh0:
                    h.wait()
                for h in sh1:
                    h.wait()
                return carry

            lax.fori_loop(0, ns // 2, pair, 0)

        @pl.when(c == 0)
        def _():
            run_edges(t0)

        @pl.when(c == 1)
        def _():
            run_edges(t1)

        plsc.subcore_barrier()
        pltpu.sync_copy(acc.at[pl.ds(base, ROWS_PER_TILE)],
                        out.at[pl.ds(c * N + base, ROWS_PER_TILE)])

    return seg


_seg_edge = _make_segsum(True)
_seg_feat = _make_segsum(False)


def _emb_body(x8_ref, w_ref, o_ref):
    o_ref[...] = jnp.dot(x8_ref[...], w_ref[...],
                         preferred_element_type=jnp.float32)


_emb = pl.pallas_call(
    _emb_body,
    out_shape=jax.ShapeDtypeStruct((NP, 128), jnp.float32),
)


def _layer0_body(a_ref, bd0_ref, bd1_ref, o0_ref, o1_ref):
    a = a_ref[:NP, :] + a_ref[NP:, :]
    o0_ref[...] = jnp.maximum(
        jnp.dot(a, bd0_ref[...], preferred_element_type=jnp.float32), 0.0)
    o1_ref[...] = jnp.maximum(
        jnp.dot(a, bd1_ref[...], preferred_element_type=jnp.float32), 0.0)


_layer0 = pl.pallas_call(
    _layer0_body,
    out_shape=[
        jax.ShapeDtypeStruct((NP, 128), jnp.float32),
        jax.ShapeDtypeStruct((NP, 128), jnp.float32),
    ],
)


def _mid_body(a_ref, r0_ref, r1_ref,
              bd00_ref, bd01_ref, bd10_ref, bd11_ref, o0_ref, o1_ref):
    a0 = a_ref[:NP, :]
    a1 = a_ref[NP:, :]
    h0 = jnp.dot(a0, bd00_ref[...], preferred_element_type=jnp.float32)
    h0 += jnp.dot(a1, bd10_ref[...], preferred_element_type=jnp.float32)
    h1 = jnp.dot(a0, bd01_ref[...], preferred_element_type=jnp.float32)
    h1 += jnp.dot(a1, bd11_ref[...], preferred_element_type=jnp.float32)
    o0_ref[...] = jnp.maximum(h0, 0.0) + r0_ref[...]
    o1_ref[...] = jnp.maximum(h1, 0.0) + r1_ref[...]


_mid = pl.pallas_call(
    _mid_body,
    out_shape=[
        jax.ShapeDtypeStruct((NP, 128), jnp.float32),
        jax.ShapeDtypeStruct((NP, 128), jnp.float32),
    ],
)


def _final_body(a_ref, r0_ref, r1_ref,
                bd00_ref, bd01_ref, bd10_ref, bd11_ref,
                onode_ref, g_ref):
    a0 = a_ref[:NP, :]
    a1 = a_ref[NP:, :]
    h0 = jnp.dot(a0, bd00_ref[...], preferred_element_type=jnp.float32)
    h0 += jnp.dot(a1, bd10_ref[...], preferred_element_type=jnp.float32)
    h1 = jnp.dot(a0, bd01_ref[...], preferred_element_type=jnp.float32)
    h1 += jnp.dot(a1, bd11_ref[...], preferred_element_type=jnp.float32)
    h0 = jnp.maximum(h0, 0.0) + r0_ref[...]
    h1 = jnp.maximum(h1, 0.0) + r1_ref[...]
    onode_ref[0] = h0
    onode_ref[1] = h1
    g_ref[...] = jnp.stack([jnp.sum(h0, axis=0), jnp.sum(h1, axis=0)])


_final = pl.pallas_call(
    _final_body,
    out_shape=[
        jax.ShapeDtypeStruct((2, NP, 128), jnp.float32),
        jax.ShapeDtypeStruct((2, 128), jnp.float32),
    ],
)


_ILV_B = 2504


def _ilv_body(h_ref, o_ref):
    h0 = h_ref[0]
    h1 = h_ref[1]
    pieces = []
    for a in range(8):
        pieces.append(h0[:, a * HW:(a + 1) * HW])
        pieces.append(h1[:, a * HW:(a + 1) * HW])
    o_ref[...] = jnp.concatenate(pieces, axis=1)


_ilv = pl.pallas_call(
    _ilv_body,
    grid=((NP + _ILV_B - 1) // _ILV_B,),
    in_specs=[pl.BlockSpec((2, _ILV_B, 128), lambda i: (0, i, 0))],
    out_specs=pl.BlockSpec((_ILV_B, 256), lambda i: (i, 0)),
    out_shape=jax.ShapeDtypeStruct((NP, 256), jnp.float32),
)


def _kron8(w):
    return jnp.kron(jnp.eye(8, dtype=w.dtype), w)


def kernel(x, edge_index, W_emb, W0, W1, W2):
    er = edge_index.reshape(2, NTILES, NG, NSUB, S)
    x8 = x.reshape(NP, 8 * D_N)

    W_embp = jnp.pad(W_emb, ((0, 0), (0, HW - H_IN)))
    W_embb = jnp.kron(jnp.eye(8, dtype=x.dtype), W_embp)
    W0p = jnp.pad(W0, ((0, HW - H_IN), (0, 0)))

    bdl0 = [_kron8(W0p[:, :HW]), _kron8(W0p[:, HW:])]
    bd1 = [[_kron8(W1[i * HW:(i + 1) * HW, j * HW:(j + 1) * HW])
            for j in (0, 1)] for i in (0, 1)]
    bd2 = [[_kron8(W2[i * HW:(i + 1) * HW, j * HW:(j + 1) * HW])
            for j in (0, 1)] for i in (0, 1)]

    h0 = _emb(x8, W_embb)
    h0f = h0.reshape(N, HW)
    A0 = _seg_edge(h0f, h0f, er)
    h1a, h1b = _layer0(A0.reshape(2 * NP, 128), *bdl0)
    A1 = _seg_feat(h1a.reshape(N, HW), h1b.reshape(N, HW), er)
    h2a, h2b = _mid(A1.reshape(2 * NP, 128), h1a, h1b,
                    bd1[0][0], bd1[0][1], bd1[1][0], bd1[1][1])
    A2 = _seg_feat(h2a.reshape(N, HW), h2b.reshape(N, HW), er)
    h3, gcols = _final(A2.reshape(2 * NP, 128), h2a, h2b,
                       bd2[0][0], bd2[0][1], bd2[1][0], bd2[1][1])
    emb_node = _ilv(h3).reshape(N, EMB)
    emb_graph = gcols.reshape(2, 8, HW).sum(axis=1).reshape(1, EMB)
    return emb_node, emb_graph


# speedup vs baseline: 1.1022x; 1.1022x over previous
import functools

import jax
import jax.numpy as jnp
from jax import lax
from jax.experimental import pallas as pl
from jax.experimental.pallas import tpu as pltpu
from jax.experimental.pallas import tpu_sc as plsc

N = 100000
E = 1600000
D_N = 27
H_IN = 8
EMB = 32
HW = 16
NP = N // 8

NTILES = 16
NCORES = 2
ROWS_PER_TILE = N // NTILES

S = 125
NSUB = 5
G = S * NSUB
NG = E // (NTILES * G)
NG_HALF = NG // 2


def _make_segsum(edge_split: bool):
    ns = NG_HALF if edge_split else NG
    goff = 1 if edge_split else 0
    w = H_IN if edge_split else HW
    mesh = plsc.VectorSubcoreMesh(core_axis_name="c", subcore_axis_name="s")

    @functools.partial(
        pl.kernel,
        out_type=jax.ShapeDtypeStruct((NCORES * N, w), jnp.float32),
        mesh=mesh,
        compiler_params=pltpu.CompilerParams(use_tc_tiling_on_sc=False),
        scratch_types=[
            pltpu.VMEM((2, NSUB, S), jnp.int32),
            pltpu.VMEM((2, NSUB, S), jnp.int32),
            pltpu.VMEM((2, G, w), jnp.float32),
            pltpu.MemorySpace.VMEM_SHARED((N, w), jnp.float32),
            pltpu.SemaphoreType.DMA,
            pltpu.SemaphoreType.DMA,
        ],
    )
    def seg(t0, t1, er, zt, out, sidx, didx, rows, acc, gsem, ssem):
        c = lax.axis_index("c")
        s = lax.axis_index("s")
        base = s * ROWS_PER_TILE

        pltpu.sync_copy(zt, acc.at[pl.ds(base, ROWS_PER_TILE)])
        plsc.subcore_barrier()

        def run_edges(tref):
            g0 = c * (ns * goff)

            def pair(p, carry):
                g = g0 + 2 * p
                pltpu.sync_copy(er.at[0, s, g], sidx.at[0])
                pltpu.sync_copy(er.at[1, s, g], didx.at[0])
                gh0 = [
                    pltpu.async_copy(tref.at[sidx.at[0, j]],
                                     rows.at[0, pl.ds(j * S, S)], gsem)
                    for j in range(NSUB)
                ]
                pltpu.sync_copy(er.at[0, s, g + 1], sidx.at[1])
                pltpu.sync_copy(er.at[1, s, g + 1], didx.at[1])
                gh1 = [
                    pltpu.async_copy(tref.at[sidx.at[1, j]],
                                     rows.at[1, pl.ds(j * S, S)], gsem)
                    for j in range(NSUB)
                ]
                for h in gh0:
                    h.wait()
                sh0 = [
                    pltpu.async_copy(rows.at[0, pl.ds(j * S, S)],
                                     acc.at[didx.at[0, j]], ssem, add=True)
                    for j in range(NSUB)
                ]
                for h in gh1:
                    h.wait()
                sh1 = [
                    pltpu.async_copy(rows.at[1, pl.ds(j * S, S)],
                                     acc.at[didx.at[1, j]], ssem, add=True)
                    for j in range(NSUB)
                ]
                for h in sh0:
                    h.wait()
                for h in sh1:
                    h.wait()
                return carry

            lax.fori_loop(0, ns // 2, pair, 0)

        @pl.when(c == 0)
        def _():
            run_edges(t0)

        @pl.when(c == 1)
        def _():
            run_edges(t1)

        plsc.subcore_barrier()
        pltpu.sync_copy(acc.at[pl.ds(base, ROWS_PER_TILE)],
                        out.at[pl.ds(c * N + base, ROWS_PER_TILE)])

    return seg


_seg_edge = _make_segsum(True)
_seg_feat = _make_segsum(False)


N16 = N // 16


def _emb_body(x16_ref, w_ref, o_ref):
    o_ref[...] = jnp.dot(x16_ref[...], w_ref[...],
                         preferred_element_type=jnp.float32)


_emb = pl.pallas_call(
    _emb_body,
    out_shape=jax.ShapeDtypeStruct((N16, 128), jnp.float32),
)


def _layer0_body(a_ref, bd0_ref, bd1_ref, o0_ref, o1_ref):
    a = a_ref[:N16, :] + a_ref[N16:, :]
    o0_ref[...] = jnp.maximum(
        jnp.dot(a, bd0_ref[...], preferred_element_type=jnp.float32), 0.0)
    o1_ref[...] = jnp.maximum(
        jnp.dot(a, bd1_ref[...], preferred_element_type=jnp.float32), 0.0)


_layer0 = pl.pallas_call(
    _layer0_body,
    out_shape=[
        jax.ShapeDtypeStruct((N16, 256), jnp.float32),
        jax.ShapeDtypeStruct((N16, 256), jnp.float32),
    ],
)


def _mid_body(a_ref, r0_ref, r1_ref,
              bd00_ref, bd01_ref, bd10_ref, bd11_ref, o0_ref, o1_ref):
    a0 = a_ref[:NP, :]
    a1 = a_ref[NP:, :]
    h0 = jnp.dot(a0, bd00_ref[...], preferred_element_type=jnp.float32)
    h0 += jnp.dot(a1, bd10_ref[...], preferred_element_type=jnp.float32)
    h1 = jnp.dot(a0, bd01_ref[...], preferred_element_type=jnp.float32)
    h1 += jnp.dot(a1, bd11_ref[...], preferred_element_type=jnp.float32)
    o0_ref[...] = jnp.maximum(h0, 0.0) + r0_ref[...]
    o1_ref[...] = jnp.maximum(h1, 0.0) + r1_ref[...]


_mid = pl.pallas_call(
    _mid_body,
    out_shape=[
        jax.ShapeDtypeStruct((NP, 128), jnp.float32),
        jax.ShapeDtypeStruct((NP, 128), jnp.float32),
    ],
)


def _final_body(a_ref, r0_ref, r1_ref,
                bd00_ref, bd01_ref, bd10_ref, bd11_ref,
                onode_ref, g_ref):
    a0 = a_ref[:NP, :]
    a1 = a_ref[NP:, :]
    h0 = jnp.dot(a0, bd00_ref[...], preferred_element_type=jnp.float32)
    h0 += jnp.dot(a1, bd10_ref[...], preferred_element_type=jnp.float32)
    h1 = jnp.dot(a0, bd01_ref[...], preferred_element_type=jnp.float32)
    h1 += jnp.dot(a1, bd11_ref[...], preferred_element_type=jnp.float32)
    h0 = jnp.maximum(h0, 0.0) + r0_ref[...]
    h1 = jnp.maximum(h1, 0.0) + r1_ref[...]
    onode_ref[0] = h0
    onode_ref[1] = h1
    g_ref[...] = jnp.stack([jnp.sum(h0, axis=0), jnp.sum(h1, axis=0)])


_final = pl.pallas_call(
    _final_body,
    out_shape=[
        jax.ShapeDtypeStruct((2, NP, 128), jnp.float32),
        jax.ShapeDtypeStruct((2, 128), jnp.float32),
    ],
)


_ILV_B = 2504


def _ilv_body(h_ref, o_ref):
    h0 = h_ref[0]
    h1 = h_ref[1]
    pieces = []
    for a in range(8):
        pieces.append(h0[:, a * HW:(a + 1) * HW])
        pieces.append(h1[:, a * HW:(a + 1) * HW])
    o_ref[...] = jnp.concatenate(pieces, axis=1)


_ilv = pl.pallas_call(
    _ilv_body,
    grid=((NP + _ILV_B - 1) // _ILV_B,),
    in_specs=[pl.BlockSpec((2, _ILV_B, 128), lambda i: (0, i, 0))],
    out_specs=pl.BlockSpec((_ILV_B, 256), lambda i: (i, 0)),
    out_shape=jax.ShapeDtypeStruct((NP, 256), jnp.float32),
)


def _kron8(w):
    return jnp.kron(jnp.eye(8, dtype=w.dtype), w)


def kernel(x, edge_index, W_emb, W0, W1, W2):
    er = edge_index.reshape(2, NTILES, NG, NSUB, S)
    x16 = x.reshape(N16, 16 * D_N)

    W_embb = jnp.kron(jnp.eye(16, dtype=x.dtype), W_emb)
    eye16 = jnp.eye(16, dtype=x.dtype)
    bdl0 = [jnp.kron(eye16, W0[:, :HW]), jnp.kron(eye16, W0[:, HW:])]
    z8 = jnp.zeros((ROWS_PER_TILE, H_IN), jnp.float32)
    z16 = jnp.zeros((ROWS_PER_TILE, HW), jnp.float32)
    bd1 = [[_kron8(W1[i * HW:(i + 1) * HW, j * HW:(j + 1) * HW])
            for j in (0, 1)] for i in (0, 1)]
    bd2 = [[_kron8(W2[i * HW:(i + 1) * HW, j * HW:(j + 1) * HW])
            for j in (0, 1)] for i in (0, 1)]

    h0 = _emb(x16, W_embb)
    h0f = h0.reshape(N, H_IN)
    A0 = _seg_edge(h0f, h0f, er, z8)
    h1a, h1b = _layer0(A0.reshape(2 * N16, 128), *bdl0)
    A1 = _seg_feat(h1a.reshape(N, HW), h1b.reshape(N, HW), er, z16)
    h1ap = h1a.reshape(NP, 128)
    h1bp = h1b.reshape(NP, 128)
    h2a, h2b = _mid(A1.reshape(2 * NP, 128), h1ap, h1bp,
                    bd1[0][0], bd1[0][1], bd1[1][0], bd1[1][1])
    A2 = _seg_feat(h2a.reshape(N, HW), h2b.reshape(N, HW), er, z16)
    h3, gcols = _final(A2.reshape(2 * NP, 128), h2a, h2b,
                       bd2[0][0], bd2[0][1], bd2[1][0], bd2[1][1])
    emb_node = _ilv(h3).reshape(N, EMB)
    emb_graph = gcols.reshape(2, 8, HW).sum(axis=1).reshape(1, EMB)
    return emb_node, emb_graph


# speedup vs baseline: 1.1387x; 1.0331x over previous
import functools

import jax
import jax.numpy as jnp
from jax import lax
from jax.experimental import pallas as pl
from jax.experimental.pallas import tpu as pltpu
from jax.experimental.pallas import tpu_sc as plsc

N = 100000
E = 1600000
D_N = 27
H_IN = 8
EMB = 32
HW = 16
NP = N // 8

NTILES = 16
NCORES = 2
ROWS_PER_TILE = N // NTILES

S = 100
NSUB = 5
G = S * NSUB
NG = E // (NTILES * G)
NG_HALF = NG // 2


def _make_segsum(edge_split: bool):
    ns = NG_HALF if edge_split else NG
    goff = 1 if edge_split else 0
    mesh = plsc.VectorSubcoreMesh(core_axis_name="c", subcore_axis_name="s")

    @functools.partial(
        pl.kernel,
        out_type=jax.ShapeDtypeStruct((NCORES * N, HW), jnp.float32),
        mesh=mesh,
        compiler_params=pltpu.CompilerParams(use_tc_tiling_on_sc=False),
        scratch_types=[
            pltpu.VMEM((3, NSUB, S), jnp.int32),
            pltpu.VMEM((3, NSUB, S), jnp.int32),
            pltpu.VMEM((3, G, HW), jnp.float32),
            pltpu.MemorySpace.VMEM_SHARED((N, HW), jnp.float32),
            pltpu.SemaphoreType.DMA,
            pltpu.SemaphoreType.DMA,
        ],
    )
    def seg(t0, t1, er, out, sidx, didx, rows, acc, gsem, ssem):
        c = lax.axis_index("c")
        s = lax.axis_index("s")
        base = s * ROWS_PER_TILE

        zvec = jnp.zeros((16,), jnp.float32)

        def zero_row(i, carry):
            rows[0, i, :] = zvec
            return carry

        lax.fori_loop(0, G, zero_row, 0)
        for k in range(ROWS_PER_TILE // G):
            pltpu.sync_copy(rows.at[0], acc.at[pl.ds(base + k * G, G)])
        rem = ROWS_PER_TILE % G
        if rem:
            pltpu.sync_copy(rows.at[0, pl.ds(0, rem)],
                            acc.at[pl.ds(base + ROWS_PER_TILE - rem, rem)])
        plsc.subcore_barrier()

        def run_edges(tref):
            g0 = c * (ns * goff)

            def triple(q, carry):
                g = g0 + 3 * q
                for b in range(3):
                    @pl.when(q >= 1)
                    def _():
                        for j in range(NSUB):
                            pltpu.make_async_copy(
                                rows.at[b, pl.ds(j * S, S)],
                                acc.at[didx.at[b, j]], ssem).wait()
                    pltpu.sync_copy(er.at[0, s, g + b], sidx.at[b])
                    pltpu.sync_copy(er.at[1, s, g + b], didx.at[b])
                    for j in range(NSUB):
                        pltpu.async_copy(tref.at[sidx.at[b, j]],
                                         rows.at[b, pl.ds(j * S, S)], gsem)
                for b in range(3):
                    for j in range(NSUB):
                        pltpu.make_async_copy(
                            tref.at[sidx.at[b, j]],
                            rows.at[b, pl.ds(j * S, S)], gsem).wait()
                    for j in range(NSUB):
                        pltpu.async_copy(rows.at[b, pl.ds(j * S, S)],
                                         acc.at[didx.at[b, j]], ssem,
                                         add=True)
                return carry

            nt = ns // 3
            lax.fori_loop(0, nt, triple, 0)
            for j in range(NSUB):
                pltpu.make_async_copy(rows.at[0, pl.ds(j * S, S)],
                                      acc.at[didx.at[0, j]], ssem).wait()
                pltpu.make_async_copy(rows.at[1, pl.ds(j * S, S)],
                                      acc.at[didx.at[1, j]], ssem).wait()
                pltpu.make_async_copy(rows.at[2, pl.ds(j * S, S)],
                                      acc.at[didx.at[2, j]], ssem).wait()
            for k in range(ns - 3 * nt):
                g = g0 + 3 * nt + k
                pltpu.sync_copy(er.at[0, s, g], sidx.at[0])
                pltpu.sync_copy(er.at[1, s, g], didx.at[0])
                gh = [
                    pltpu.async_copy(tref.at[sidx.at[0, j]],
                                     rows.at[0, pl.ds(j * S, S)], gsem)
                    for j in range(NSUB)
                ]
                for h in gh:
                    h.wait()
                sh = [
                    pltpu.async_copy(rows.at[0, pl.ds(j * S, S)],
                                     acc.at[didx.at[0, j]], ssem, add=True)
                    for j in range(NSUB)
                ]
                for h in sh:
                    h.wait()

        @pl.when(c == 0)
        def _():
            run_edges(t0)

        @pl.when(c == 1)
        def _():
            run_edges(t1)

        plsc.subcore_barrier()
        pltpu.sync_copy(acc.at[pl.ds(base, ROWS_PER_TILE)],
                        out.at[pl.ds(c * N + base, ROWS_PER_TILE)])

    return seg


_seg_edge = _make_segsum(True)
_seg_feat = _make_segsum(False)


def _emb_body(x8_ref, w_ref, o_ref):
    o_ref[...] = jnp.dot(x8_ref[...], w_ref[...],
                         preferred_element_type=jnp.float32)


_emb = pl.pallas_call(
    _emb_body,
    out_shape=jax.ShapeDtypeStruct((NP, 128), jnp.float32),
)


def _layer0_body(a_ref, bd0_ref, bd1_ref, o0_ref, o1_ref):
    a = a_ref[:NP, :] + a_ref[NP:, :]
    o0_ref[...] = jnp.maximum(
        jnp.dot(a, bd0_ref[...], preferred_element_type=jnp.float32), 0.0)
    o1_ref[...] = jnp.maximum(
        jnp.dot(a, bd1_ref[...], preferred_element_type=jnp.float32), 0.0)


_layer0 = pl.pallas_call(
    _layer0_body,
    out_shape=[
        jax.ShapeDtypeStruct((NP, 128), jnp.float32),
        jax.ShapeDtypeStruct((NP, 128), jnp.float32),
    ],
)


def _mid_body(a_ref, r0_ref, r1_ref,
              bd00_ref, bd01_ref, bd10_ref, bd11_ref, o0_ref, o1_ref):
    a0 = a_ref[:NP, :]
    a1 = a_ref[NP:, :]
    h0 = jnp.dot(a0, bd00_ref[...], preferred_element_type=jnp.float32)
    h0 += jnp.dot(a1, bd10_ref[...], preferred_element_type=jnp.float32)
    h1 = jnp.dot(a0, bd01_ref[...], preferred_element_type=jnp.float32)
    h1 += jnp.dot(a1, bd11_ref[...], preferred_element_type=jnp.float32)
    o0_ref[...] = jnp.maximum(h0, 0.0) + r0_ref[...]
    o1_ref[...] = jnp.maximum(h1, 0.0) + r1_ref[...]


_mid = pl.pallas_call(
    _mid_body,
    out_shape=[
        jax.ShapeDtypeStruct((NP, 128), jnp.float32),
        jax.ShapeDtypeStruct((NP, 128), jnp.float32),
    ],
)


def _final_body(a_ref, r0_ref, r1_ref,
                bd00_ref, bd01_ref, bd10_ref, bd11_ref,
                onode_ref, g_ref):
    a0 = a_ref[:NP, :]
    a1 = a_ref[NP:, :]
    h0 = jnp.dot(a0, bd00_ref[...], preferred_element_type=jnp.float32)
    h0 += jnp.dot(a1, bd10_ref[...], preferred_element_type=jnp.float32)
    h1 = jnp.dot(a0, bd01_ref[...], preferred_element_type=jnp.float32)
    h1 += jnp.dot(a1, bd11_ref[...], preferred_element_type=jnp.float32)
    h0 = jnp.maximum(h0, 0.0) + r0_ref[...]
    h1 = jnp.maximum(h1, 0.0) + r1_ref[...]
    onode_ref[0] = h0
    onode_ref[1] = h1
    g_ref[...] = jnp.stack([jnp.sum(h0, axis=0), jnp.sum(h1, axis=0)])


_final = pl.pallas_call(
    _final_body,
    out_shape=[
        jax.ShapeDtypeStruct((2, NP, 128), jnp.float32),
        jax.ShapeDtypeStruct((2, 128), jnp.float32),
    ],
)


_ILV_B = 2504


def _ilv_body(h_ref, o_ref):
    h0 = h_ref[0]
    h1 = h_ref[1]
    pieces = []
    for a in range(8):
        pieces.append(h0[:, a * HW:(a + 1) * HW])
        pieces.append(h1[:, a * HW:(a + 1) * HW])
    o_ref[...] = jnp.concatenate(pieces, axis=1)


_ilv = pl.pallas_call(
    _ilv_body,
    grid=((NP + _ILV_B - 1) // _ILV_B,),
    in_specs=[pl.BlockSpec((2, _ILV_B, 128), lambda i: (0, i, 0))],
    out_specs=pl.BlockSpec((_ILV_B, 256), lambda i: (i, 0)),
    out_shape=jax.ShapeDtypeStruct((NP, 256), jnp.float32),
)


def _kron8(w):
    return jnp.kron(jnp.eye(8, dtype=w.dtype), w)


def kernel(x, edge_index, W_emb, W0, W1, W2):
    er = edge_index.reshape(2, NTILES, NG, NSUB, S)
    x8 = x.reshape(NP, 8 * D_N)

    W_embp = jnp.pad(W_emb, ((0, 0), (0, HW - H_IN)))
    W_embb = jnp.kron(jnp.eye(8, dtype=x.dtype), W_embp)
    W0p = jnp.pad(W0, ((0, HW - H_IN), (0, 0)))

    bdl0 = [_kron8(W0p[:, :HW]), _kron8(W0p[:, HW:])]
    bd1 = [[_kron8(W1[i * HW:(i + 1) * HW, j * HW:(j + 1) * HW])
            for j in (0, 1)] for i in (0, 1)]
    bd2 = [[_kron8(W2[i * HW:(i + 1) * HW, j * HW:(j + 1) * HW])
            for j in (0, 1)] for i in (0, 1)]

    h0 = _emb(x8, W_embb)
    h0f = h0.reshape(N, HW)
    A0 = _seg_edge(h0f, h0f, er)
    h1a, h1b = _layer0(A0.reshape(2 * NP, 128), *bdl0)
    A1 = _seg_feat(h1a.reshape(N, HW), h1b.reshape(N, HW), er)
    h2a, h2b = _mid(A1.reshape(2 * NP, 128), h1a, h1b,
                    bd1[0][0], bd1[0][1], bd1[1][0], bd1[1][1])
    A2 = _seg_feat(h2a.reshape(N, HW), h2b.reshape(N, HW), er)
    h3, gcols = _final(A2.reshape(2 * NP, 128), h2a, h2b,
                       bd2[0][0], bd2[0][1], bd2[1][0], bd2[1][1])
    emb_node = _ilv(h3).reshape(N, EMB)
    emb_graph = gcols.reshape(2, 8, HW).sum(axis=1).reshape(1, EMB)
    return emb_node, emb_graph
